# baseline (speedup 1.0000x reference)
import functools

import jax
import jax.numpy as jnp
from jax import lax
from jax.experimental import pallas as pl
from jax.experimental.pallas import tpu as pltpu
from jax.experimental.pallas import tpu_sc as plsc

_N = 10000
_E = 160000
_D1 = 256
_D2 = 64
_NC, _NS, _L = 2, 16, 16
_EPT = _E // _NS
_CHUNK = 80
_NCHUNK = _EPT // _CHUNK
_HALF = _N // _NC
_PADH = 5120
_NPAD = 10240
_BN = 1000


_EPT32 = _E // (_NC * _NS)
_DCH = 100
_DNCH = _EPT32 // _DCH
_DW = 16


def _make_deg():
    mesh = plsc.VectorSubcoreMesh(core_axis_name="c", subcore_axis_name="s")
    stripe = _NPAD // _NS

    @functools.partial(
        pl.kernel, mesh=mesh,
        compiler_params=pltpu.CompilerParams(use_tc_tiling_on_sc=False),
        out_type=jax.ShapeDtypeStruct((_NC, _NPAD, _DW), jnp.float32),
        scratch_types=[
            pltpu.VMEM((_DNCH, _DCH), jnp.int32),
            pltpu.VMEM((_DCH, _DW), jnp.float32),
            pltpu.VMEM((stripe, _DW), jnp.float32),
            pltpu.VMEM_SHARED((_NPAD, _DW), jnp.float32),
        ],
    )
    def degk(dst_hbm, deg_out, didx_v, ones_v, zbuf_v, acc_sh):
        c = lax.axis_index("c")
        s = lax.axis_index("s")
        gid = c * _NS + s
        pltpu.sync_copy(dst_hbm.at[gid], didx_v)
        of = jnp.ones((_L,), jnp.float32)
        zf = jnp.zeros((_L,), jnp.float32)

        def fill_body(i, _):
            ones_v[i, :] = of
            return _
        lax.fori_loop(0, _DCH, fill_body, None)

        def zfill_body(i, _):
            zbuf_v[i, :] = zf
            return _
        lax.fori_loop(0, stripe, zfill_body, None)
        pltpu.sync_copy(zbuf_v, acc_sh.at[pl.ds(s * stripe, stripe)])
        plsc.subcore_barrier()

        def e_body(j, _):
            pltpu.sync_copy(ones_v, acc_sh.at[didx_v.at[j]], add=True)
            return _
        lax.fori_loop(0, _DNCH, e_body, None)
        plsc.subcore_barrier()

        pltpu.sync_copy(acc_sh.at[pl.ds(s * stripe, stripe)],
                        deg_out.at[c, pl.ds(s * stripe, stripe)])

    return degk


_EPAD = 163840
_EPP = _EPAD // _NS
_CH = 128
_NCHMAX = _EPP // _CH
_CROWS = _NCHMAX + 1
_CFLAT = _CROWS * _CH


def _make_smooth(FB, NF):
    mesh = plsc.VectorSubcoreMesh(core_axis_name="c", subcore_axis_name="s")
    rows_per_tile = _PADH // _NS
    wout = 312

    out_type = [jax.ShapeDtypeStruct((_N, FB), jnp.float32)] * NF

    @functools.partial(
        pl.kernel, mesh=mesh,
        compiler_params=pltpu.CompilerParams(use_tc_tiling_on_sc=False,
                                             needs_layout_passes=False),
        out_type=out_type,
        scratch_types=[
            pltpu.VMEM((_EPP,), jnp.int32),
            pltpu.VMEM((_EPP,), jnp.int32),
            pltpu.VMEM((_CFLAT,), jnp.int32),
            pltpu.VMEM((_CFLAT,), jnp.int32),
            pltpu.VMEM((_CROWS, _CH), jnp.int32),
            pltpu.VMEM((_CH, FB), jnp.float32),
            pltpu.VMEM((_CH, FB), jnp.float32),
            pltpu.VMEM((_CH, FB), jnp.float32),
            pltpu.VMEM_SHARED((_PADH, FB), jnp.float32),
            pltpu.SemaphoreType.DMA,
            pltpu.SemaphoreType.DMA,
        ],
    )
    def smooth(*refs):
        h_refs = refs[:NF]
        src_hbm, dst_hbm = refs[NF], refs[NF + 1]
        out_refs = refs[NF + 2:2 * NF + 2]
        (src_v, dst_v, csrc_f, cldst_f, cldst2, rows_a, rows_b, zbuf_v,
         acc_sh, sem_a, sem_b) = refs[2 * NF + 2:]
        c = lax.axis_index("c")
        s = lax.axis_index("s")
        pltpu.sync_copy(src_hbm.at[s], src_v)
        pltpu.sync_copy(dst_hbm.at[s], dst_v)
        nbase = c * _HALF

        zi = jnp.zeros((_L,), jnp.int32)
        gi = jnp.full((_L,), _HALF, jnp.int32)

        def pre_body(i, _):
            csrc_f[pl.ds(i * _L, _L)] = zi
            cldst_f[pl.ds(i * _L, _L)] = gi
            return _
        lax.fori_loop(0, _CFLAT // _L, pre_body, None)

        def comp_body(i, cnt):
            d = dst_v[pl.ds(i * _L, _L)]
            sg = src_v[pl.ds(i * _L, _L)]
            ld = d - nbase
            ok = (ld >= 0) & (ld < _HALF)
            plsc.store_compressed(csrc_f.at[pl.ds(cnt, _L)], sg, mask=ok)
            plsc.store_compressed(cldst_f.at[pl.ds(cnt, _L)], ld, mask=ok)
            return cnt + plsc.all_reduce_population_count(ok)[0]
        cnt = lax.fori_loop(0, _EPP // _L, comp_body, jnp.int32(0))

        def f2d_body(r, _):
            for q in range(_CH // _L):
                cldst2[r, pl.ds(q * _L, _L)] = cldst_f[pl.ds(r * _CH + q * _L, _L)]
            return _
        lax.fori_loop(0, _CROWS, f2d_body, None)

        nch = (cnt + (_CH - 1)) // _CH
        npair = nch // 2

        zf = jnp.zeros((_L,), jnp.float32)

        def zrow_body(i, _):
            for q in range(FB // _L):
                zbuf_v[i, pl.ds(q * _L, _L)] = zf
            return _
        lax.fori_loop(0, _CH, zrow_body, None)

        for f in range(NF):
            if f > 0:
                plsc.subcore_barrier()
            for off, sz in ((0, _CH), (_CH, _CH), (2 * _CH, rows_per_tile - 2 * _CH)):
                pltpu.sync_copy(
                    zbuf_v.at[pl.ds(0, sz)],
                    acc_sh.at[pl.ds(s * rows_per_tile + off, sz)])
            plsc.subcore_barrier()

            h = h_refs[f]

            def _gather(j, buf, sem):
                pltpu.async_copy(h.at[csrc_f.at[pl.ds(j * _CH, _CH)]], buf, sem)

            def _gwait(j, buf, sem):
                pltpu.make_async_copy(
                    h.at[csrc_f.at[pl.ds(j * _CH, _CH)]], buf, sem).wait()

            def _scat(j, buf):
                pltpu.sync_copy(buf, acc_sh.at[cldst2.at[j]], add=True)

            _gather(0, rows_a, sem_a)

            def pair_body(p, _):
                j0 = p * 2
                _gwait(j0, rows_a, sem_a)
                _gather(j0 + 1, rows_b, sem_b)
                _scat(j0, rows_a)
                _gwait(j0 + 1, rows_b, sem_b)
                _gather(j0 + 2, rows_a, sem_a)
                _scat(j0 + 1, rows_b)
                return _
            lax.fori_loop(0, npair, pair_body, None)
            _gwait(2 * npair, rows_a, sem_a)
            _scat(2 * npair, rows_a)
            plsc.subcore_barrier()

            pltpu.sync_copy(acc_sh.at[pl.ds(s * wout, wout)],
                            out_refs[f].at[pl.ds(c * _HALF + s * wout, wout)])

            @pl.when(s == _NS - 1)
            def _(f=f):
                rem = _HALF - _NS * wout
                pltpu.sync_copy(acc_sh.at[pl.ds(_NS * wout, rem)],
                                out_refs[f].at[pl.ds(c * _HALF + _NS * wout, rem)])

    return smooth


_deg_kernel = _make_deg()
_FB = 64
_NF1 = _D1 // _FB
_smooth_d1 = _make_smooth(_FB, _NF1)
_smooth_d2 = _make_smooth(_D2, 1)


def _tc1(X, W1, b1, dega, degb):
    def body(x_ref, w_ref, b_ref, da_ref, db_ref,
             h_ref, *out_refs):
        hp_refs = out_refs[:_NF1]
        isq_ref, inv_ref = out_refs[_NF1], out_refs[_NF1 + 1]
        d1 = da_ref[...] + db_ref[...] + 1.0
        isq = lax.rsqrt(d1)
        h = jnp.dot(x_ref[...], w_ref[...],
                    preferred_element_type=jnp.float32) + b_ref[...]
        hp = h * isq
        h_ref[...] = h
        for k in range(_NF1):
            hp_refs[k][...] = hp[:, k * _FB:(k + 1) * _FB]
        isq_ref[...] = isq
        inv_ref[...] = 1.0 / d1

    return pl.pallas_call(
        body, grid=(_N // _BN,),
        in_specs=[
            pl.BlockSpec((_BN, _D1), lambda i: (i, 0)),
            pl.BlockSpec((_D1, _D1), lambda i: (0, 0)),
            pl.BlockSpec((1, _D1), lambda i: (0, 0)),
            pl.BlockSpec((_BN, 1), lambda i: (i, 0)),
            pl.BlockSpec((_BN, 1), lambda i: (i, 0)),
        ],
        out_specs=(
            [pl.BlockSpec((_BN, _D1), lambda i: (i, 0))]
            + [pl.BlockSpec((_BN, _FB), lambda i: (i, 0))] * _NF1
            + [pl.BlockSpec((_BN, 1), lambda i: (i, 0))] * 2
        ),
        out_shape=(
            [jax.ShapeDtypeStruct((_N, _D1), jnp.float32)]
            + [jax.ShapeDtypeStruct((_N, _FB), jnp.float32)] * _NF1
            + [jax.ShapeDtypeStruct((_N, 1), jnp.float32)] * 2
        ),
    )(X, W1, b1, dega, degb)


def _tc2(S1s, h1, isq, inv, W2p, b2p):
    def body(*refs):
        s1_refs = refs[:_NF1]
        (h1_ref, isq_ref, inv_ref, w_ref, b_ref,
         o1_ref, h2_ref, hp_ref) = refs[_NF1:]
        isq = isq_ref[...]
        s1 = jnp.concatenate([r[...] for r in s1_refs], axis=1)
        out1 = jnp.maximum(isq * s1 + inv_ref[...] * h1_ref[...], 0.0)
        h2 = jnp.dot(out1, w_ref[...],
                     preferred_element_type=jnp.float32) + b_ref[...]
        o1_ref[...] = out1
        h2_ref[...] = h2
        hp_ref[...] = h2 * isq

    return pl.pallas_call(
        body, grid=(_N // _BN,),
        in_specs=[pl.BlockSpec((_BN, _FB), lambda i: (i, 0))] * _NF1 + [
            pl.BlockSpec((_BN, _D1), lambda i: (i, 0)),
            pl.BlockSpec((_BN, 1), lambda i: (i, 0)),
            pl.BlockSpec((_BN, 1), lambda i: (i, 0)),
            pl.BlockSpec((_D1, _D2), lambda i: (0, 0)),
            pl.BlockSpec((1, _D2), lambda i: (0, 0)),
        ],
        out_specs=[
            pl.BlockSpec((_BN, _D1), lambda i: (i, 0)),
            pl.BlockSpec((_BN, _D2), lambda i: (i, 0)),
            pl.BlockSpec((_BN, _D2), lambda i: (i, 0)),
        ],
        out_shape=[
            jax.ShapeDtypeStruct((_N, _D1), jnp.float32),
            jax.ShapeDtypeStruct((_N, _D2), jnp.float32),
            jax.ShapeDtypeStruct((_N, _D2), jnp.float32),
        ],
    )(*S1s, h1, isq, inv, W2p, b2p)


def _tc3(S2, h2, isq, inv):
    def body(s2_ref, h2_ref, isq_ref, inv_ref, o2_ref):
        o2_ref[...] = isq_ref[...] * s2_ref[...] + inv_ref[...] * h2_ref[...]

    return pl.pallas_call(
        body, grid=(_N // _BN,),
        in_specs=[
            pl.BlockSpec((_BN, _D2), lambda i: (i, 0)),
            pl.BlockSpec((_BN, _D2), lambda i: (i, 0)),
            pl.BlockSpec((_BN, 1), lambda i: (i, 0)),
            pl.BlockSpec((_BN, 1), lambda i: (i, 0)),
        ],
        out_specs=pl.BlockSpec((_BN, _D2), lambda i: (i, 0)),
        out_shape=jax.ShapeDtypeStruct((_N, _D2), jnp.float32),
    )(S2, h2, isq, inv)


def kernel(X, edge_index, W1, b1, W2, b2):
    pad = _EPAD - _E
    src = jnp.concatenate(
        [edge_index[0], jnp.zeros((pad,), jnp.int32)]).reshape(_NS, _EPP)
    dst = jnp.concatenate(
        [edge_index[1], jnp.full((pad,), _N, jnp.int32)]).reshape(_NS, _EPP)
    dst_deg = edge_index[1].reshape(_NC * _NS, _DNCH, _DCH)

    degP = _deg_kernel(dst_deg)
    dega = degP[0, :_N, 0].reshape(_N, 1)
    degb = degP[1, :_N, 0].reshape(_N, 1)

    h1, *rest = _tc1(X, W1, b1.reshape(1, _D1), dega, degb)
    h1ps, isq, inv = rest[:_NF1], rest[_NF1], rest[_NF1 + 1]
    S1s = _smooth_d1(*h1ps, src, dst)

    W2p = jnp.pad(W2, ((0, 0), (0, _D2 - W2.shape[1])))
    b2p = jnp.pad(b2, (0, _D2 - b2.shape[0])).reshape(1, _D2)
    out1, h2, h2p = _tc2(S1s, h1, isq, inv, W2p, b2p)

    (S2,) = _smooth_d2(h2p, src, dst)
    out2p = _tc3(S2, h2, isq, inv)
    return (out1, out2p[:, :W2.shape[1]])


# speedup vs baseline: 1.1207x; 1.1207x over previous
import functools

import jax
import jax.numpy as jnp
from jax import lax
from jax.experimental import pallas as pl
from jax.experimental.pallas import tpu as pltpu
from jax.experimental.pallas import tpu_sc as plsc

_N = 10000
_E = 160000
_D1 = 256
_D2 = 64
_NC, _NS, _L = 2, 16, 16
_EPT = _E // _NS
_CHUNK = 80
_NCHUNK = _EPT // _CHUNK
_HALF = _N // _NC
_PADH = 5120
_NPAD = 10240
_BN = 1000


_EPT32 = _E // (_NC * _NS)
_DCH = 100
_DNCH = _EPT32 // _DCH
_DW = 16

_EPAD = 163840
_EPP = _EPAD // _NS
_CH = 128
_NCHMAX = _EPP // _CH
_CROWS = _NCHMAX + 1
_CFLAT = _CROWS * _CH


def _make_deg():
    mesh = plsc.VectorSubcoreMesh(core_axis_name="c", subcore_axis_name="s")
    stripe = _NPAD // _NS

    @functools.partial(
        pl.kernel, mesh=mesh,
        compiler_params=pltpu.CompilerParams(use_tc_tiling_on_sc=False,
                                             needs_layout_passes=False),
        out_type=[
            jax.ShapeDtypeStruct((_NC, _NPAD, _DW), jnp.float32),
            jax.ShapeDtypeStruct((_NC, _NS, _CFLAT), jnp.int32),
            jax.ShapeDtypeStruct((_NC, _NS, _CROWS, _CH), jnp.int32),
            jax.ShapeDtypeStruct((_NC, _NS, _L), jnp.int32),
        ],
        scratch_types=[
            pltpu.VMEM((_DNCH, _DCH), jnp.int32),
            pltpu.VMEM((_DCH, _DW), jnp.float32),
            pltpu.VMEM((stripe, _DW), jnp.float32),
            pltpu.VMEM((_EPP,), jnp.int32),
            pltpu.VMEM((_EPP,), jnp.int32),
            pltpu.VMEM((_CFLAT,), jnp.int32),
            pltpu.VMEM((_CFLAT,), jnp.int32),
            pltpu.VMEM((_CROWS, _CH), jnp.int32),
            pltpu.VMEM((_L,), jnp.int32),
            pltpu.VMEM_SHARED((_NPAD, _DW), jnp.float32),
        ],
    )
    def degk(dst32_hbm, src16_hbm, dst16_hbm,
             deg_out, csrc_out, cldst_out, cnt_out,
             didx_v, ones_v, zbuf_v, src_v, dst_v, csrc_f, cldst_f, cldst2,
             cnt_v, acc_sh):
        c = lax.axis_index("c")
        s = lax.axis_index("s")
        gid = c * _NS + s
        pltpu.sync_copy(dst32_hbm.at[gid], didx_v)
        pltpu.sync_copy(src16_hbm.at[s], src_v)
        pltpu.sync_copy(dst16_hbm.at[s], dst_v)
        of = jnp.ones((_L,), jnp.float32)
        zf = jnp.zeros((_L,), jnp.float32)

        def fill_body(i, _):
            ones_v[i, :] = of
            return _
        lax.fori_loop(0, _DCH, fill_body, None)

        def zfill_body(i, _):
            zbuf_v[i, :] = zf
            return _
        lax.fori_loop(0, stripe, zfill_body, None)
        pltpu.sync_copy(zbuf_v, acc_sh.at[pl.ds(s * stripe, stripe)])
        plsc.subcore_barrier()

        def e_body(j, _):
            pltpu.sync_copy(ones_v, acc_sh.at[didx_v.at[j]], add=True)
            return _
        lax.fori_loop(0, _DNCH, e_body, None)

        nbase = c * _HALF
        zi = jnp.zeros((_L,), jnp.int32)
        gi = jnp.full((_L,), _HALF, jnp.int32)

        def pre_body(i, _):
            csrc_f[pl.ds(i * _L, _L)] = zi
            cldst_f[pl.ds(i * _L, _L)] = gi
            return _
        lax.fori_loop(0, _CFLAT // _L, pre_body, None)

        def comp_body(i, cnt):
            d = dst_v[pl.ds(i * _L, _L)]
            sg = src_v[pl.ds(i * _L, _L)]
            ld = d - nbase
            ok = (ld >= 0) & (ld < _HALF)
            plsc.store_compressed(csrc_f.at[pl.ds(cnt, _L)], sg, mask=ok)
            plsc.store_compressed(cldst_f.at[pl.ds(cnt, _L)], ld, mask=ok)
            return cnt + plsc.all_reduce_population_count(ok)[0]
        cnt = lax.fori_loop(0, _EPP // _L, comp_body, jnp.int32(0))

        def f2d_body(r, _):
            for q in range(_CH // _L):
                cldst2[r, pl.ds(q * _L, _L)] = cldst_f[pl.ds(r * _CH + q * _L, _L)]
            return _
        lax.fori_loop(0, _CROWS, f2d_body, None)

        cnt_v[:] = jnp.broadcast_to(cnt, (_L,))
        pltpu.sync_copy(csrc_f, csrc_out.at[c, s])
        pltpu.sync_copy(cldst2, cldst_out.at[c, s])
        pltpu.sync_copy(cnt_v, cnt_out.at[c, s])

        plsc.subcore_barrier()
        pltpu.sync_copy(acc_sh.at[pl.ds(s * stripe, stripe)],
                        deg_out.at[c, pl.ds(s * stripe, stripe)])

    return degk


def _make_smooth(FB, NF):
    mesh = plsc.VectorSubcoreMesh(core_axis_name="c", subcore_axis_name="s")
    rows_per_tile = _PADH // _NS
    wout = 312

    out_type = [jax.ShapeDtypeStruct((_N, FB), jnp.float32)] * NF

    @functools.partial(
        pl.kernel, mesh=mesh,
        compiler_params=pltpu.CompilerParams(use_tc_tiling_on_sc=False),
        out_type=out_type,
        scratch_types=[
            pltpu.VMEM((_CFLAT,), jnp.int32),
            pltpu.VMEM((_CROWS, _CH), jnp.int32),
            pltpu.VMEM((_L,), jnp.int32),
            pltpu.VMEM((_CH, FB), jnp.float32),
            pltpu.VMEM((_CH, FB), jnp.float32),
            pltpu.VMEM((_CH, FB), jnp.float32),
            pltpu.VMEM_SHARED((_PADH, FB), jnp.float32),
            pltpu.SemaphoreType.DMA,
            pltpu.SemaphoreType.DMA,
        ],
    )
    def smooth(*refs):
        h_refs = refs[:NF]
        csrc_hbm, cldst_hbm, cnt_hbm = refs[NF], refs[NF + 1], refs[NF + 2]
        out_refs = refs[NF + 3:2 * NF + 3]
        (csrc_f, cldst2, cnt_v, rows_a, rows_b, zbuf_v,
         acc_sh, sem_a, sem_b) = refs[2 * NF + 3:]
        c = lax.axis_index("c")
        s = lax.axis_index("s")
        pltpu.sync_copy(csrc_hbm.at[c, s], csrc_f)
        pltpu.sync_copy(cldst_hbm.at[c, s], cldst2)
        pltpu.sync_copy(cnt_hbm.at[c, s], cnt_v)
        cnt = cnt_v[:][0]

        nch = (cnt + (_CH - 1)) // _CH
        npair = nch // 2

        zf = jnp.zeros((_L,), jnp.float32)

        def zrow_body(i, _):
            for q in range(FB // _L):
                zbuf_v[i, pl.ds(q * _L, _L)] = zf
            return _
        lax.fori_loop(0, _CH, zrow_body, None)

        for f in range(NF):
            if f > 0:
                plsc.subcore_barrier()
            for off, sz in ((0, _CH), (_CH, _CH), (2 * _CH, rows_per_tile - 2 * _CH)):
                pltpu.sync_copy(
                    zbuf_v.at[pl.ds(0, sz)],
                    acc_sh.at[pl.ds(s * rows_per_tile + off, sz)])
            plsc.subcore_barrier()

            h = h_refs[f]

            def _gather(j, buf, sem):
                pltpu.async_copy(h.at[csrc_f.at[pl.ds(j * _CH, _CH)]], buf, sem)

            def _gwait(j, buf, sem):
                pltpu.make_async_copy(
                    h.at[csrc_f.at[pl.ds(j * _CH, _CH)]], buf, sem).wait()

            def _scat(j, buf):
                pltpu.sync_copy(buf, acc_sh.at[cldst2.at[j]], add=True)

            _gather(0, rows_a, sem_a)

            def pair_body(p, _):
                j0 = p * 2
                _gwait(j0, rows_a, sem_a)
                _gather(j0 + 1, rows_b, sem_b)
                _scat(j0, rows_a)
                _gwait(j0 + 1, rows_b, sem_b)
                _gather(j0 + 2, rows_a, sem_a)
                _scat(j0 + 1, rows_b)
                return _
            lax.fori_loop(0, npair, pair_body, None)
            _gwait(2 * npair, rows_a, sem_a)
            _scat(2 * npair, rows_a)
            plsc.subcore_barrier()

            pltpu.sync_copy(acc_sh.at[pl.ds(s * wout, wout)],
                            out_refs[f].at[pl.ds(c * _HALF + s * wout, wout)])

            @pl.when(s == _NS - 1)
            def _(f=f):
                rem = _HALF - _NS * wout
                pltpu.sync_copy(acc_sh.at[pl.ds(_NS * wout, rem)],
                                out_refs[f].at[pl.ds(c * _HALF + _NS * wout, rem)])

    return smooth


_deg_kernel = _make_deg()
_FB = 128
_NF1 = _D1 // _FB
_smooth_d1 = _make_smooth(_FB, _NF1)
_smooth_d2 = _make_smooth(_D2, 1)


def _tc1(X, W1, b1, dega, degb):
    def body(x_ref, w_ref, b_ref, da_ref, db_ref,
             h_ref, *out_refs):
        hp_refs = out_refs[:_NF1]
        isq_ref, inv_ref = out_refs[_NF1], out_refs[_NF1 + 1]
        d1 = da_ref[...] + db_ref[...] + 1.0
        isq = lax.rsqrt(d1)
        h = jnp.dot(x_ref[...], w_ref[...],
                    preferred_element_type=jnp.float32) + b_ref[...]
        hp = h * isq
        h_ref[...] = h
        for k in range(_NF1):
            hp_refs[k][...] = hp[:, k * _FB:(k + 1) * _FB]
        isq_ref[...] = isq
        inv_ref[...] = 1.0 / d1

    return pl.pallas_call(
        body, grid=(_N // _BN,),
        in_specs=[
            pl.BlockSpec((_BN, _D1), lambda i: (i, 0)),
            pl.BlockSpec((_D1, _D1), lambda i: (0, 0)),
            pl.BlockSpec((1, _D1), lambda i: (0, 0)),
            pl.BlockSpec((_BN, 1), lambda i: (i, 0)),
            pl.BlockSpec((_BN, 1), lambda i: (i, 0)),
        ],
        out_specs=(
            [pl.BlockSpec((_BN, _D1), lambda i: (i, 0))]
            + [pl.BlockSpec((_BN, _FB), lambda i: (i, 0))] * _NF1
            + [pl.BlockSpec((_BN, 1), lambda i: (i, 0))] * 2
        ),
        out_shape=(
            [jax.ShapeDtypeStruct((_N, _D1), jnp.float32)]
            + [jax.ShapeDtypeStruct((_N, _FB), jnp.float32)] * _NF1
            + [jax.ShapeDtypeStruct((_N, 1), jnp.float32)] * 2
        ),
    )(X, W1, b1, dega, degb)


def _tc2(S1s, h1, isq, inv, W2p, b2p):
    def body(*refs):
        s1_refs = refs[:_NF1]
        (h1_ref, isq_ref, inv_ref, w_ref, b_ref,
         o1_ref, h2_ref, hp_ref) = refs[_NF1:]
        isq = isq_ref[...]
        s1 = jnp.concatenate([r[...] for r in s1_refs], axis=1)
        out1 = jnp.maximum(isq * s1 + inv_ref[...] * h1_ref[...], 0.0)
        h2 = jnp.dot(out1, w_ref[...],
                     preferred_element_type=jnp.float32) + b_ref[...]
        o1_ref[...] = out1
        h2_ref[...] = h2
        hp_ref[...] = h2 * isq

    return pl.pallas_call(
        body, grid=(_N // _BN,),
        in_specs=[pl.BlockSpec((_BN, _FB), lambda i: (i, 0))] * _NF1 + [
            pl.BlockSpec((_BN, _D1), lambda i: (i, 0)),
            pl.BlockSpec((_BN, 1), lambda i: (i, 0)),
            pl.BlockSpec((_BN, 1), lambda i: (i, 0)),
            pl.BlockSpec((_D1, _D2), lambda i: (0, 0)),
            pl.BlockSpec((1, _D2), lambda i: (0, 0)),
        ],
        out_specs=[
            pl.BlockSpec((_BN, _D1), lambda i: (i, 0)),
            pl.BlockSpec((_BN, _D2), lambda i: (i, 0)),
            pl.BlockSpec((_BN, _D2), lambda i: (i, 0)),
        ],
        out_shape=[
            jax.ShapeDtypeStruct((_N, _D1), jnp.float32),
            jax.ShapeDtypeStruct((_N, _D2), jnp.float32),
            jax.ShapeDtypeStruct((_N, _D2), jnp.float32),
        ],
    )(*S1s, h1, isq, inv, W2p, b2p)


def _tc3(S2, h2, isq, inv):
    def body(s2_ref, h2_ref, isq_ref, inv_ref, o2_ref):
        o2_ref[...] = isq_ref[...] * s2_ref[...] + inv_ref[...] * h2_ref[...]

    return pl.pallas_call(
        body, grid=(_N // _BN,),
        in_specs=[
            pl.BlockSpec((_BN, _D2), lambda i: (i, 0)),
            pl.BlockSpec((_BN, _D2), lambda i: (i, 0)),
            pl.BlockSpec((_BN, 1), lambda i: (i, 0)),
            pl.BlockSpec((_BN, 1), lambda i: (i, 0)),
        ],
        out_specs=pl.BlockSpec((_BN, _D2), lambda i: (i, 0)),
        out_shape=jax.ShapeDtypeStruct((_N, _D2), jnp.float32),
    )(S2, h2, isq, inv)


def kernel(X, edge_index, W1, b1, W2, b2):
    pad = _EPAD - _E
    src = jnp.concatenate(
        [edge_index[0], jnp.zeros((pad,), jnp.int32)]).reshape(_NS, _EPP)
    dst = jnp.concatenate(
        [edge_index[1], jnp.full((pad,), _N, jnp.int32)]).reshape(_NS, _EPP)
    dst_deg = edge_index[1].reshape(_NC * _NS, _DNCH, _DCH)

    degP, csrc, cldst, cnts = _deg_kernel(dst_deg, src, dst)
    dega = degP[0, :_N, 0].reshape(_N, 1)
    degb = degP[1, :_N, 0].reshape(_N, 1)

    h1, *rest = _tc1(X, W1, b1.reshape(1, _D1), dega, degb)
    h1ps, isq, inv = rest[:_NF1], rest[_NF1], rest[_NF1 + 1]
    S1s = _smooth_d1(*h1ps, csrc, cldst, cnts)

    W2p = jnp.pad(W2, ((0, 0), (0, _D2 - W2.shape[1])))
    b2p = jnp.pad(b2, (0, _D2 - b2.shape[0])).reshape(1, _D2)
    out1, h2, h2p = _tc2(S1s, h1, isq, inv, W2p, b2p)

    (S2,) = _smooth_d2(h2p, csrc, cldst, cnts)
    out2p = _tc3(S2, h2, isq, inv)
    return (out1, out2p[:, :W2.shape[1]])


# speedup vs baseline: 1.1210x; 1.0002x over previous
import functools

import jax
import jax.numpy as jnp
from jax import lax
from jax.experimental import pallas as pl
from jax.experimental.pallas import tpu as pltpu
from jax.experimental.pallas import tpu_sc as plsc

_N = 10000
_E = 160000
_D1 = 256
_D2 = 64
_NC, _NS, _L = 2, 16, 16
_EPT = _E // _NS
_CHUNK = 80
_NCHUNK = _EPT // _CHUNK
_HALF = _N // _NC
_PADH = 5120
_NPAD = 10240
_BN = 1000


_EPT32 = _E // (_NC * _NS)
_DCH = 100
_DNCH = _EPT32 // _DCH
_DW = 16

_EPAD = 163840
_EPP = _EPAD // _NS
_CH = 128
_NCHMAX = _EPP // _CH
_CROWS = _NCHMAX + 1
_CFLAT = _CROWS * _CH


def _make_deg():
    mesh = plsc.VectorSubcoreMesh(core_axis_name="c", subcore_axis_name="s")
    stripe = _NPAD // _NS

    @functools.partial(
        pl.kernel, mesh=mesh,
        compiler_params=pltpu.CompilerParams(use_tc_tiling_on_sc=False,
                                             needs_layout_passes=False),
        out_type=[
            jax.ShapeDtypeStruct((_NC, _NPAD, _DW), jnp.float32),
            jax.ShapeDtypeStruct((_NC, _NS, _CROWS, _CH), jnp.int32),
            jax.ShapeDtypeStruct((_NC, _NS, _CROWS, _CH), jnp.int32),
            jax.ShapeDtypeStruct((_NC, _NS, _L), jnp.int32),
        ],
        scratch_types=[
            pltpu.VMEM((_DNCH, _DCH), jnp.int32),
            pltpu.VMEM((_DCH, _DW), jnp.float32),
            pltpu.VMEM((stripe, _DW), jnp.float32),
            pltpu.VMEM((_EPP,), jnp.int32),
            pltpu.VMEM((_EPP,), jnp.int32),
            pltpu.VMEM((_CFLAT,), jnp.int32),
            pltpu.VMEM((_CFLAT,), jnp.int32),
            pltpu.VMEM((_CROWS, _CH), jnp.int32),
            pltpu.VMEM((_CROWS, _CH), jnp.int32),
            pltpu.VMEM((_L,), jnp.int32),
            pltpu.VMEM_SHARED((_NPAD, _DW), jnp.float32),
        ],
    )
    def degk(dst32_hbm, src16_hbm, dst16_hbm,
             deg_out, csrc_out, cldst_out, cnt_out,
             didx_v, ones_v, zbuf_v, src_v, dst_v, csrc_f, cldst_f, csrc2,
             cldst2, cnt_v, acc_sh):
        c = lax.axis_index("c")
        s = lax.axis_index("s")
        gid = c * _NS + s
        pltpu.sync_copy(dst32_hbm.at[gid], didx_v)
        pltpu.sync_copy(src16_hbm.at[s], src_v)
        pltpu.sync_copy(dst16_hbm.at[s], dst_v)
        of = jnp.ones((_L,), jnp.float32)
        zf = jnp.zeros((_L,), jnp.float32)

        def fill_body(i, _):
            ones_v[i, :] = of
            return _
        lax.fori_loop(0, _DCH, fill_body, None)

        def zfill_body(i, _):
            zbuf_v[i, :] = zf
            return _
        lax.fori_loop(0, stripe, zfill_body, None)
        pltpu.sync_copy(zbuf_v, acc_sh.at[pl.ds(s * stripe, stripe)])
        plsc.subcore_barrier()

        def e_body(j, _):
            pltpu.sync_copy(ones_v, acc_sh.at[didx_v.at[j]], add=True)
            return _
        lax.fori_loop(0, _DNCH, e_body, None)

        nbase = c * _HALF
        zi = jnp.zeros((_L,), jnp.int32)
        gi = jnp.full((_L,), _HALF, jnp.int32)

        def pre_body(i, _):
            csrc_f[pl.ds(i * _L, _L)] = zi
            cldst_f[pl.ds(i * _L, _L)] = gi
            return _
        lax.fori_loop(0, _CFLAT // _L, pre_body, None)

        def comp_body(i, cnt):
            d = dst_v[pl.ds(i * _L, _L)]
            sg = src_v[pl.ds(i * _L, _L)]
            ld = d - nbase
            ok = (ld >= 0) & (ld < _HALF)
            plsc.store_compressed(csrc_f.at[pl.ds(cnt, _L)], sg, mask=ok)
            plsc.store_compressed(cldst_f.at[pl.ds(cnt, _L)], ld, mask=ok)
            return cnt + plsc.all_reduce_population_count(ok)[0]
        cnt = lax.fori_loop(0, _EPP // _L, comp_body, jnp.int32(0))

        def f2d_body(r, _):
            for q in range(_CH // _L):
                sl = pl.ds(r * _CH + q * _L, _L)
                csrc2[r, pl.ds(q * _L, _L)] = csrc_f[sl]
                cldst2[r, pl.ds(q * _L, _L)] = cldst_f[sl]
            return _
        lax.fori_loop(0, _CROWS, f2d_body, None)

        cnt_v[:] = jnp.broadcast_to(cnt, (_L,))
        pltpu.sync_copy(csrc2, csrc_out.at[c, s])
        pltpu.sync_copy(cldst2, cldst_out.at[c, s])
        pltpu.sync_copy(cnt_v, cnt_out.at[c, s])

        plsc.subcore_barrier()
        pltpu.sync_copy(acc_sh.at[pl.ds(s * stripe, stripe)],
                        deg_out.at[c, pl.ds(s * stripe, stripe)])

    return degk


def _make_smooth(FB, NF):
    mesh = plsc.VectorSubcoreMesh(core_axis_name="c", subcore_axis_name="s")
    rows_per_tile = _PADH // _NS
    wout = 312

    out_type = [jax.ShapeDtypeStruct((_N, FB), jnp.float32)] * NF

    @functools.partial(
        pl.kernel, mesh=mesh,
        compiler_params=pltpu.CompilerParams(use_tc_tiling_on_sc=False),
        out_type=out_type,
        scratch_types=[
            pltpu.VMEM((_CROWS, _CH), jnp.int32),
            pltpu.VMEM((_CROWS, _CH), jnp.int32),
            pltpu.VMEM((_L,), jnp.int32),
            pltpu.VMEM((_CH, FB), jnp.float32),
            pltpu.VMEM((_CH, FB), jnp.float32),
            pltpu.VMEM((_CH, FB), jnp.float32),
            pltpu.VMEM_SHARED((_PADH, FB), jnp.float32),
            pltpu.SemaphoreType.DMA,
            pltpu.SemaphoreType.DMA,
        ],
    )
    def smooth(*refs):
        h_refs = refs[:NF]
        csrc_hbm, cldst_hbm, cnt_hbm = refs[NF], refs[NF + 1], refs[NF + 2]
        out_refs = refs[NF + 3:2 * NF + 3]
        (csrc2, cldst2, cnt_v, rows_a, rows_b, zbuf_v,
         acc_sh, sem_a, sem_b) = refs[2 * NF + 3:]
        c = lax.axis_index("c")
        s = lax.axis_index("s")
        pltpu.sync_copy(csrc_hbm.at[c, s], csrc2)
        pltpu.sync_copy(cldst_hbm.at[c, s], cldst2)
        pltpu.sync_copy(cnt_hbm.at[c, s], cnt_v)
        cnt = cnt_v[:][0]

        nch = (cnt + (_CH - 1)) // _CH
        npair = nch // 2

        zf = jnp.zeros((_L,), jnp.float32)

        def zrow_body(i, _):
            for q in range(FB // _L):
                zbuf_v[i, pl.ds(q * _L, _L)] = zf
            return _
        lax.fori_loop(0, _CH, zrow_body, None)

        for f in range(NF):
            if f > 0:
                plsc.subcore_barrier()
            for off, sz in ((0, _CH), (_CH, _CH), (2 * _CH, rows_per_tile - 2 * _CH)):
                pltpu.sync_copy(
                    zbuf_v.at[pl.ds(0, sz)],
                    acc_sh.at[pl.ds(s * rows_per_tile + off, sz)])
            plsc.subcore_barrier()

            h = h_refs[f]

            def _gather(j, buf, sem):
                pltpu.async_copy(h.at[csrc2.at[j]], buf, sem)

            def _gwait(j, buf, sem):
                pltpu.make_async_copy(h.at[csrc2.at[j]], buf, sem).wait()

            def _scat(j, buf):
                pltpu.sync_copy(buf, acc_sh.at[cldst2.at[j]], add=True)

            _gather(0, rows_a, sem_a)

            def pair_body(p, _):
                j0 = p * 2
                _gwait(j0, rows_a, sem_a)
                _gather(j0 + 1, rows_b, sem_b)
                _scat(j0, rows_a)
                _gwait(j0 + 1, rows_b, sem_b)
                _gather(j0 + 2, rows_a, sem_a)
                _scat(j0 + 1, rows_b)
                return _
            lax.fori_loop(0, npair, pair_body, None)
            _gwait(2 * npair, rows_a, sem_a)
            _scat(2 * npair, rows_a)
            plsc.subcore_barrier()

            pltpu.sync_copy(acc_sh.at[pl.ds(s * wout, wout)],
                            out_refs[f].at[pl.ds(c * _HALF + s * wout, wout)])

            @pl.when(s == _NS - 1)
            def _(f=f):
                rem = _HALF - _NS * wout
                pltpu.sync_copy(acc_sh.at[pl.ds(_NS * wout, rem)],
                                out_refs[f].at[pl.ds(c * _HALF + _NS * wout, rem)])

    return smooth


_deg_kernel = _make_deg()
_FB = 128
_NF1 = _D1 // _FB
_smooth_d1 = _make_smooth(_FB, _NF1)
_smooth_d2 = _make_smooth(_D2, 1)


def _tc1(X, W1, b1, dega, degb):
    def body(x_ref, w_ref, b_ref, da_ref, db_ref,
             h_ref, *out_refs):
        hp_refs = out_refs[:_NF1]
        isq_ref, inv_ref = out_refs[_NF1], out_refs[_NF1 + 1]
        d1 = da_ref[...] + db_ref[...] + 1.0
        isq = lax.rsqrt(d1)
        h = jnp.dot(x_ref[...], w_ref[...],
                    preferred_element_type=jnp.float32) + b_ref[...]
        hp = h * isq
        h_ref[...] = h
        for k in range(_NF1):
            hp_refs[k][...] = hp[:, k * _FB:(k + 1) * _FB]
        isq_ref[...] = isq
        inv_ref[...] = 1.0 / d1

    return pl.pallas_call(
        body, grid=(_N // _BN,),
        in_specs=[
            pl.BlockSpec((_BN, _D1), lambda i: (i, 0)),
            pl.BlockSpec((_D1, _D1), lambda i: (0, 0)),
            pl.BlockSpec((1, _D1), lambda i: (0, 0)),
            pl.BlockSpec((_BN, 1), lambda i: (i, 0)),
            pl.BlockSpec((_BN, 1), lambda i: (i, 0)),
        ],
        out_specs=(
            [pl.BlockSpec((_BN, _D1), lambda i: (i, 0))]
            + [pl.BlockSpec((_BN, _FB), lambda i: (i, 0))] * _NF1
            + [pl.BlockSpec((_BN, 1), lambda i: (i, 0))] * 2
        ),
        out_shape=(
            [jax.ShapeDtypeStruct((_N, _D1), jnp.float32)]
            + [jax.ShapeDtypeStruct((_N, _FB), jnp.float32)] * _NF1
            + [jax.ShapeDtypeStruct((_N, 1), jnp.float32)] * 2
        ),
    )(X, W1, b1, dega, degb)


def _tc2(S1s, h1, isq, inv, W2p, b2p):
    def body(*refs):
        s1_refs = refs[:_NF1]
        (h1_ref, isq_ref, inv_ref, w_ref, b_ref,
         o1_ref, h2_ref, hp_ref) = refs[_NF1:]
        isq = isq_ref[...]
        s1 = jnp.concatenate([r[...] for r in s1_refs], axis=1)
        out1 = jnp.maximum(isq * s1 + inv_ref[...] * h1_ref[...], 0.0)
        h2 = jnp.dot(out1, w_ref[...],
                     preferred_element_type=jnp.float32) + b_ref[...]
        o1_ref[...] = out1
        h2_ref[...] = h2
        hp_ref[...] = h2 * isq

    return pl.pallas_call(
        body, grid=(_N // _BN,),
        in_specs=[pl.BlockSpec((_BN, _FB), lambda i: (i, 0))] * _NF1 + [
            pl.BlockSpec((_BN, _D1), lambda i: (i, 0)),
            pl.BlockSpec((_BN, 1), lambda i: (i, 0)),
            pl.BlockSpec((_BN, 1), lambda i: (i, 0)),
            pl.BlockSpec((_D1, _D2), lambda i: (0, 0)),
            pl.BlockSpec((1, _D2), lambda i: (0, 0)),
        ],
        out_specs=[
            pl.BlockSpec((_BN, _D1), lambda i: (i, 0)),
            pl.BlockSpec((_BN, _D2), lambda i: (i, 0)),
            pl.BlockSpec((_BN, _D2), lambda i: (i, 0)),
        ],
        out_shape=[
            jax.ShapeDtypeStruct((_N, _D1), jnp.float32),
            jax.ShapeDtypeStruct((_N, _D2), jnp.float32),
            jax.ShapeDtypeStruct((_N, _D2), jnp.float32),
        ],
    )(*S1s, h1, isq, inv, W2p, b2p)


def _tc3(S2, h2, isq, inv):
    def body(s2_ref, h2_ref, isq_ref, inv_ref, o2_ref):
        o2_ref[...] = isq_ref[...] * s2_ref[...] + inv_ref[...] * h2_ref[...]

    return pl.pallas_call(
        body, grid=(_N // _BN,),
        in_specs=[
            pl.BlockSpec((_BN, _D2), lambda i: (i, 0)),
            pl.BlockSpec((_BN, _D2), lambda i: (i, 0)),
            pl.BlockSpec((_BN, 1), lambda i: (i, 0)),
            pl.BlockSpec((_BN, 1), lambda i: (i, 0)),
        ],
        out_specs=pl.BlockSpec((_BN, _D2), lambda i: (i, 0)),
        out_shape=jax.ShapeDtypeStruct((_N, _D2), jnp.float32),
    )(S2, h2, isq, inv)


def kernel(X, edge_index, W1, b1, W2, b2):
    pad = _EPAD - _E
    src = jnp.concatenate(
        [edge_index[0], jnp.zeros((pad,), jnp.int32)]).reshape(_NS, _EPP)
    dst = jnp.concatenate(
        [edge_index[1], jnp.full((pad,), _N, jnp.int32)]).reshape(_NS, _EPP)
    dst_deg = edge_index[1].reshape(_NC * _NS, _DNCH, _DCH)

    degP, csrc, cldst, cnts = _deg_kernel(dst_deg, src, dst)
    dega = degP[0, :_N, 0].reshape(_N, 1)
    degb = degP[1, :_N, 0].reshape(_N, 1)

    h1, *rest = _tc1(X, W1, b1.reshape(1, _D1), dega, degb)
    h1ps, isq, inv = rest[:_NF1], rest[_NF1], rest[_NF1 + 1]
    S1s = _smooth_d1(*h1ps, csrc, cldst, cnts)

    W2p = jnp.pad(W2, ((0, 0), (0, _D2 - W2.shape[1])))
    b2p = jnp.pad(b2, (0, _D2 - b2.shape[0])).reshape(1, _D2)
    out1, h2, h2p = _tc2(S1s, h1, isq, inv, W2p, b2p)

    (S2,) = _smooth_d2(h2p, csrc, cldst, cnts)
    out2p = _tc3(S2, h2, isq, inv)
    return (out1, out2p[:, :W2.shape[1]])


# speedup vs baseline: 1.3193x; 1.1769x over previous
import functools

import jax
import jax.numpy as jnp
from jax import lax
from jax.experimental import pallas as pl
from jax.experimental.pallas import tpu as pltpu
from jax.experimental.pallas import tpu_sc as plsc

_N = 10000
_E = 160000
_D1 = 256
_D2 = 64
_NC, _NS, _L = 2, 16, 16
_EPT = _E // _NS
_CHUNK = 80
_NCHUNK = _EPT // _CHUNK
_HALF = _N // _NC
_PADH = 5120
_NPAD = 10240
_BN = 1000


_EPT32 = _E // (_NC * _NS)
_DCH = 100
_DNCH = _EPT32 // _DCH
_DW = 16

_EPAD = 163840
_EPP = _EPAD // _NS
_CH = 80
_NCHMAX = _EPP // _CH
_CROWS = _NCHMAX + 1
_CFLAT = _CROWS * _CH


def _make_deg():
    mesh = plsc.VectorSubcoreMesh(core_axis_name="c", subcore_axis_name="s")
    stripe = _NPAD // _NS

    @functools.partial(
        pl.kernel, mesh=mesh,
        compiler_params=pltpu.CompilerParams(use_tc_tiling_on_sc=False,
                                             needs_layout_passes=False),
        out_type=[
            jax.ShapeDtypeStruct((_NC, _NPAD, _DW), jnp.float32),
            jax.ShapeDtypeStruct((_NC, _NS, _CROWS, _CH), jnp.int32),
            jax.ShapeDtypeStruct((_NC, _NS, _CROWS, _CH), jnp.int32),
            jax.ShapeDtypeStruct((_NC, _NS, _L), jnp.int32),
        ],
        scratch_types=[
            pltpu.VMEM((_DNCH, _DCH), jnp.int32),
            pltpu.VMEM((_DCH, _DW), jnp.float32),
            pltpu.VMEM((stripe, _DW), jnp.float32),
            pltpu.VMEM((_EPP,), jnp.int32),
            pltpu.VMEM((_EPP,), jnp.int32),
            pltpu.VMEM((_CFLAT,), jnp.int32),
            pltpu.VMEM((_CFLAT,), jnp.int32),
            pltpu.VMEM((_CROWS, _CH), jnp.int32),
            pltpu.VMEM((_CROWS, _CH), jnp.int32),
            pltpu.VMEM((_L,), jnp.int32),
            pltpu.VMEM_SHARED((_NPAD, _DW), jnp.float32),
        ],
    )
    def degk(dst32_hbm, src16_hbm, dst16_hbm,
             deg_out, csrc_out, cldst_out, cnt_out,
             didx_v, ones_v, zbuf_v, src_v, dst_v, csrc_f, cldst_f, csrc2,
             cldst2, cnt_v, acc_sh):
        c = lax.axis_index("c")
        s = lax.axis_index("s")
        gid = c * _NS + s
        pltpu.sync_copy(dst32_hbm.at[gid], didx_v)
        pltpu.sync_copy(src16_hbm.at[s], src_v)
        pltpu.sync_copy(dst16_hbm.at[s], dst_v)
        of = jnp.ones((_L,), jnp.float32)
        zf = jnp.zeros((_L,), jnp.float32)

        def fill_body(i, _):
            ones_v[i, :] = of
            return _
        lax.fori_loop(0, _DCH, fill_body, None)

        def zfill_body(i, _):
            zbuf_v[i, :] = zf
            return _
        lax.fori_loop(0, stripe, zfill_body, None)
        pltpu.sync_copy(zbuf_v, acc_sh.at[pl.ds(s * stripe, stripe)])
        plsc.subcore_barrier()

        def e_body(j, _):
            pltpu.sync_copy(ones_v, acc_sh.at[didx_v.at[j]], add=True)
            return _
        lax.fori_loop(0, _DNCH, e_body, None)

        nbase = c * _HALF
        zi = jnp.zeros((_L,), jnp.int32)
        gi = jnp.full((_L,), _HALF, jnp.int32)

        def pre_body(i, _):
            csrc_f[pl.ds(i * _L, _L)] = zi
            cldst_f[pl.ds(i * _L, _L)] = gi
            return _
        lax.fori_loop(0, _CFLAT // _L, pre_body, None)

        def comp_body(i, cnt):
            d = dst_v[pl.ds(i * _L, _L)]
            sg = src_v[pl.ds(i * _L, _L)]
            ld = d - nbase
            ok = (ld >= 0) & (ld < _HALF)
            plsc.store_compressed(csrc_f.at[pl.ds(cnt, _L)], sg, mask=ok)
            plsc.store_compressed(cldst_f.at[pl.ds(cnt, _L)], ld, mask=ok)
            return cnt + plsc.all_reduce_population_count(ok)[0]
        cnt = lax.fori_loop(0, _EPP // _L, comp_body, jnp.int32(0))

        def f2d_body(r, _):
            for q in range(_CH // _L):
                sl = pl.ds(r * _CH + q * _L, _L)
                csrc2[r, pl.ds(q * _L, _L)] = csrc_f[sl]
                cldst2[r, pl.ds(q * _L, _L)] = cldst_f[sl]
            return _
        lax.fori_loop(0, _CROWS, f2d_body, None)

        cnt_v[:] = jnp.broadcast_to(cnt, (_L,))
        pltpu.sync_copy(csrc2, csrc_out.at[c, s])
        pltpu.sync_copy(cldst2, cldst_out.at[c, s])
        pltpu.sync_copy(cnt_v, cnt_out.at[c, s])

        plsc.subcore_barrier()
        pltpu.sync_copy(acc_sh.at[pl.ds(s * stripe, stripe)],
                        deg_out.at[c, pl.ds(s * stripe, stripe)])

    return degk


def _make_smooth(FB, NF):
    mesh = plsc.VectorSubcoreMesh(core_axis_name="c", subcore_axis_name="s")
    rows_per_tile = _PADH // _NS
    wout = 312

    out_type = [jax.ShapeDtypeStruct((_N, FB), jnp.float32)] * NF

    @functools.partial(
        pl.kernel, mesh=mesh,
        compiler_params=pltpu.CompilerParams(use_tc_tiling_on_sc=False),
        out_type=out_type,
        scratch_types=[
            pltpu.VMEM((_CROWS, _CH), jnp.int32),
            pltpu.VMEM((_CROWS, _CH), jnp.int32),
            pltpu.VMEM((_L,), jnp.int32),
            pltpu.VMEM((_CH, FB), jnp.float32),
            pltpu.VMEM((_CH, FB), jnp.float32),
            pltpu.VMEM((_CH, FB), jnp.float32),
            pltpu.VMEM_SHARED((_PADH, FB), jnp.float32),
            pltpu.SemaphoreType.DMA,
            pltpu.SemaphoreType.DMA,
        ],
    )
    def smooth(*refs):
        h_refs = refs[:NF]
        csrc_hbm, cldst_hbm, cnt_hbm = refs[NF], refs[NF + 1], refs[NF + 2]
        out_refs = refs[NF + 3:2 * NF + 3]
        (csrc2, cldst2, cnt_v, rows_a, rows_b, zbuf_v,
         acc_sh, sem_a, sem_b) = refs[2 * NF + 3:]
        c = lax.axis_index("c")
        s = lax.axis_index("s")
        pltpu.sync_copy(csrc_hbm.at[c, s], csrc2)
        pltpu.sync_copy(cldst_hbm.at[c, s], cldst2)
        pltpu.sync_copy(cnt_hbm.at[c, s], cnt_v)
        cnt = cnt_v[:][0]

        nch = (cnt + (_CH - 1)) // _CH
        npair = nch // 2

        zf = jnp.zeros((_L,), jnp.float32)

        def zrow_body(i, _):
            for q in range(FB // _L):
                zbuf_v[i, pl.ds(q * _L, _L)] = zf
            return _
        lax.fori_loop(0, _CH, zrow_body, None)

        for f in range(NF):
            if f > 0:
                plsc.subcore_barrier()
            for off, sz in [(k * _CH, _CH) for k in range(rows_per_tile // _CH)]:
                pltpu.sync_copy(
                    zbuf_v.at[pl.ds(0, sz)],
                    acc_sh.at[pl.ds(s * rows_per_tile + off, sz)])
            plsc.subcore_barrier()

            h = h_refs[f]

            def _gather(j, buf, sem):
                pltpu.async_copy(h.at[csrc2.at[j]], buf, sem)

            def _gwait(j, buf, sem):
                pltpu.make_async_copy(h.at[csrc2.at[j]], buf, sem).wait()

            def _scat(j, buf):
                pltpu.sync_copy(buf, acc_sh.at[cldst2.at[j]], add=True)

            _gather(0, rows_a, sem_a)

            def pair_body(p, _):
                j0 = p * 2
                _gwait(j0, rows_a, sem_a)
                _gather(j0 + 1, rows_b, sem_b)
                _scat(j0, rows_a)
                _gwait(j0 + 1, rows_b, sem_b)
                _gather(j0 + 2, rows_a, sem_a)
                _scat(j0 + 1, rows_b)
                return _
            lax.fori_loop(0, npair, pair_body, None)
            _gwait(2 * npair, rows_a, sem_a)
            _scat(2 * npair, rows_a)
            plsc.subcore_barrier()

            pltpu.sync_copy(acc_sh.at[pl.ds(s * wout, wout)],
                            out_refs[f].at[pl.ds(c * _HALF + s * wout, wout)])

            @pl.when(s == _NS - 1)
            def _(f=f):
                rem = _HALF - _NS * wout
                pltpu.sync_copy(acc_sh.at[pl.ds(_NS * wout, rem)],
                                out_refs[f].at[pl.ds(c * _HALF + _NS * wout, rem)])

    return smooth


_deg_kernel = _make_deg()
_FB = 128
_NF1 = _D1 // _FB
_smooth_d1 = _make_smooth(_FB, _NF1)
_smooth_d2 = _make_smooth(_D2, 1)


def _tc1(X, W1, b1, dega, degb):
    def body(x_ref, w_ref, b_ref, da_ref, db_ref,
             h_ref, *out_refs):
        hp_refs = out_refs[:_NF1]
        isq_ref, inv_ref = out_refs[_NF1], out_refs[_NF1 + 1]
        d1 = da_ref[...] + db_ref[...] + 1.0
        isq = lax.rsqrt(d1)
        h = jnp.dot(x_ref[...], w_ref[...],
                    preferred_element_type=jnp.float32) + b_ref[...]
        hp = h * isq
        h_ref[...] = h
        for k in range(_NF1):
            hp_refs[k][...] = hp[:, k * _FB:(k + 1) * _FB]
        isq_ref[...] = isq
        inv_ref[...] = 1.0 / d1

    return pl.pallas_call(
        body, grid=(_N // _BN,),
        in_specs=[
            pl.BlockSpec((_BN, _D1), lambda i: (i, 0)),
            pl.BlockSpec((_D1, _D1), lambda i: (0, 0)),
            pl.BlockSpec((1, _D1), lambda i: (0, 0)),
            pl.BlockSpec((_BN, 1), lambda i: (i, 0)),
            pl.BlockSpec((_BN, 1), lambda i: (i, 0)),
        ],
        out_specs=(
            [pl.BlockSpec((_BN, _D1), lambda i: (i, 0))]
            + [pl.BlockSpec((_BN, _FB), lambda i: (i, 0))] * _NF1
            + [pl.BlockSpec((_BN, 1), lambda i: (i, 0))] * 2
        ),
        out_shape=(
            [jax.ShapeDtypeStruct((_N, _D1), jnp.float32)]
            + [jax.ShapeDtypeStruct((_N, _FB), jnp.float32)] * _NF1
            + [jax.ShapeDtypeStruct((_N, 1), jnp.float32)] * 2
        ),
    )(X, W1, b1, dega, degb)


def _tc2(S1s, h1, isq, inv, W2p, b2p):
    def body(*refs):
        s1_refs = refs[:_NF1]
        (h1_ref, isq_ref, inv_ref, w_ref, b_ref,
         o1_ref, h2_ref, hp_ref) = refs[_NF1:]
        isq = isq_ref[...]
        s1 = jnp.concatenate([r[...] for r in s1_refs], axis=1)
        out1 = jnp.maximum(isq * s1 + inv_ref[...] * h1_ref[...], 0.0)
        h2 = jnp.dot(out1, w_ref[...],
                     preferred_element_type=jnp.float32) + b_ref[...]
        o1_ref[...] = out1
        h2_ref[...] = h2
        hp_ref[...] = h2 * isq

    return pl.pallas_call(
        body, grid=(_N // _BN,),
        in_specs=[pl.BlockSpec((_BN, _FB), lambda i: (i, 0))] * _NF1 + [
            pl.BlockSpec((_BN, _D1), lambda i: (i, 0)),
            pl.BlockSpec((_BN, 1), lambda i: (i, 0)),
            pl.BlockSpec((_BN, 1), lambda i: (i, 0)),
            pl.BlockSpec((_D1, _D2), lambda i: (0, 0)),
            pl.BlockSpec((1, _D2), lambda i: (0, 0)),
        ],
        out_specs=[
            pl.BlockSpec((_BN, _D1), lambda i: (i, 0)),
            pl.BlockSpec((_BN, _D2), lambda i: (i, 0)),
            pl.BlockSpec((_BN, _D2), lambda i: (i, 0)),
        ],
        out_shape=[
            jax.ShapeDtypeStruct((_N, _D1), jnp.float32),
            jax.ShapeDtypeStruct((_N, _D2), jnp.float32),
            jax.ShapeDtypeStruct((_N, _D2), jnp.float32),
        ],
    )(*S1s, h1, isq, inv, W2p, b2p)


def _tc3(S2, h2, isq, inv):
    def body(s2_ref, h2_ref, isq_ref, inv_ref, o2_ref):
        o2_ref[...] = isq_ref[...] * s2_ref[...] + inv_ref[...] * h2_ref[...]

    return pl.pallas_call(
        body, grid=(_N // _BN,),
        in_specs=[
            pl.BlockSpec((_BN, _D2), lambda i: (i, 0)),
            pl.BlockSpec((_BN, _D2), lambda i: (i, 0)),
            pl.BlockSpec((_BN, 1), lambda i: (i, 0)),
            pl.BlockSpec((_BN, 1), lambda i: (i, 0)),
        ],
        out_specs=pl.BlockSpec((_BN, _D2), lambda i: (i, 0)),
        out_shape=jax.ShapeDtypeStruct((_N, _D2), jnp.float32),
    )(S2, h2, isq, inv)


def kernel(X, edge_index, W1, b1, W2, b2):
    pad = _EPAD - _E
    src = jnp.concatenate(
        [edge_index[0], jnp.zeros((pad,), jnp.int32)]).reshape(_NS, _EPP)
    dst = jnp.concatenate(
        [edge_index[1], jnp.full((pad,), _N, jnp.int32)]).reshape(_NS, _EPP)
    dst_deg = edge_index[1].reshape(_NC * _NS, _DNCH, _DCH)

    degP, csrc, cldst, cnts = _deg_kernel(dst_deg, src, dst)
    dega = degP[0, :_N, 0].reshape(_N, 1)
    degb = degP[1, :_N, 0].reshape(_N, 1)

    h1, *rest = _tc1(X, W1, b1.reshape(1, _D1), dega, degb)
    h1ps, isq, inv = rest[:_NF1], rest[_NF1], rest[_NF1 + 1]
    S1s = _smooth_d1(*h1ps, csrc, cldst, cnts)

    W2p = jnp.pad(W2, ((0, 0), (0, _D2 - W2.shape[1])))
    b2p = jnp.pad(b2, (0, _D2 - b2.shape[0])).reshape(1, _D2)
    out1, h2, h2p = _tc2(S1s, h1, isq, inv, W2p, b2p)

    (S2,) = _smooth_d2(h2p, csrc, cldst, cnts)
    out2p = _tc3(S2, h2, isq, inv)
    return (out1, out2p[:, :W2.shape[1]])


# speedup vs baseline: 1.4367x; 1.0890x over previous
import functools

import jax
import jax.numpy as jnp
from jax import lax
from jax.experimental import pallas as pl
from jax.experimental.pallas import tpu as pltpu
from jax.experimental.pallas import tpu_sc as plsc

_N = 10000
_E = 160000
_D1 = 256
_D2 = 64
_NC, _NS, _L = 2, 16, 16
_EPT = _E // _NS
_CHUNK = 80
_NCHUNK = _EPT // _CHUNK
_HALF = _N // _NC
_PADH = 5120
_NPAD = 10240
_BN = 1000


_EPT32 = _E // (_NC * _NS)
_DCH = 100
_DNCH = _EPT32 // _DCH
_DW = 16


def _make_deg():
    mesh = plsc.VectorSubcoreMesh(core_axis_name="c", subcore_axis_name="s")
    stripe = _NPAD // _NS

    @functools.partial(
        pl.kernel, mesh=mesh,
        compiler_params=pltpu.CompilerParams(use_tc_tiling_on_sc=False),
        out_type=jax.ShapeDtypeStruct((_NC, _NPAD, _DW), jnp.float32),
        scratch_types=[
            pltpu.VMEM((_DNCH, _DCH), jnp.int32),
            pltpu.VMEM((_DCH, _DW), jnp.float32),
            pltpu.VMEM((stripe, _DW), jnp.float32),
            pltpu.VMEM_SHARED((_NPAD, _DW), jnp.float32),
        ],
    )
    def degk(dst_hbm, deg_out, didx_v, ones_v, zbuf_v, acc_sh):
        c = lax.axis_index("c")
        s = lax.axis_index("s")
        gid = c * _NS + s
        pltpu.sync_copy(dst_hbm.at[gid], didx_v)
        of = jnp.ones((_L,), jnp.float32)
        zf = jnp.zeros((_L,), jnp.float32)

        def fill_body(i, _):
            ones_v[i, :] = of
            return _
        lax.fori_loop(0, _DCH, fill_body, None)

        def zfill_body(i, _):
            zbuf_v[i, :] = zf
            return _
        lax.fori_loop(0, stripe, zfill_body, None)
        pltpu.sync_copy(zbuf_v, acc_sh.at[pl.ds(s * stripe, stripe)])
        plsc.subcore_barrier()

        def e_body(j, _):
            pltpu.sync_copy(ones_v, acc_sh.at[didx_v.at[j]], add=True)
            return _
        lax.fori_loop(0, _DNCH, e_body, None)
        plsc.subcore_barrier()

        pltpu.sync_copy(acc_sh.at[pl.ds(s * stripe, stripe)],
                        deg_out.at[c, pl.ds(s * stripe, stripe)])

    return degk


def _make_smooth(FB, NF):
    mesh = plsc.VectorSubcoreMesh(core_axis_name="c", subcore_axis_name="s")
    rows_per_tile = _PADH // _NS
    ncopy = rows_per_tile // _CHUNK
    wout = 312

    out_type = [jax.ShapeDtypeStruct((_N, FB), jnp.float32)] * NF

    @functools.partial(
        pl.kernel, mesh=mesh,
        compiler_params=pltpu.CompilerParams(use_tc_tiling_on_sc=False),
        out_type=out_type,
        scratch_types=[
            pltpu.VMEM((_NCHUNK, _CHUNK), jnp.int32),
            pltpu.VMEM((_NCHUNK, _CHUNK), jnp.int32),
            pltpu.VMEM((_NCHUNK, _CHUNK), jnp.int32),
            pltpu.VMEM((_CHUNK, FB), jnp.float32),
            pltpu.VMEM((_CHUNK, FB), jnp.float32),
            pltpu.VMEM((_CHUNK, FB), jnp.float32),
            pltpu.VMEM_SHARED((_PADH, FB), jnp.float32),
            pltpu.SemaphoreType.DMA,
            pltpu.SemaphoreType.DMA,
        ],
    )
    def smooth(*refs):
        h_refs = refs[:NF]
        src_hbm, dst_hbm = refs[NF], refs[NF + 1]
        out_refs = refs[NF + 2:2 * NF + 2]
        (src_v, dst_v, ldst_v, rows_a, rows_b, zbuf_v, acc_sh,
         sem_a, sem_b) = refs[2 * NF + 2:]
        c = lax.axis_index("c")
        s = lax.axis_index("s")
        pltpu.sync_copy(src_hbm.at[s], src_v)
        pltpu.sync_copy(dst_hbm.at[s], dst_v)
        nbase = c * _HALF

        def ld_body(r, _):
            for q in range(_CHUNK // _L):
                d = dst_v[r, pl.ds(q * _L, _L)]
                ldv = d - nbase
                ok = (ldv >= 0) & (ldv < _HALF)
                ldst_v[r, pl.ds(q * _L, _L)] = jnp.where(ok, ldv, _HALF)
            return _
        lax.fori_loop(0, _NCHUNK, ld_body, None)

        zf = jnp.zeros((_L,), jnp.float32)

        def zrow_body(i, _):
            for q in range(FB // _L):
                zbuf_v[i, pl.ds(q * _L, _L)] = zf
            return _
        lax.fori_loop(0, _CHUNK, zrow_body, None)

        for f in range(NF):
            if f > 0:
                plsc.subcore_barrier()
            for k in range(ncopy):
                pltpu.sync_copy(
                    zbuf_v,
                    acc_sh.at[pl.ds(s * rows_per_tile + k * _CHUNK, _CHUNK)])
            plsc.subcore_barrier()

            h = h_refs[f]

            def _gather(j, buf, sem):
                pltpu.async_copy(h.at[src_v.at[j]], buf, sem)

            def _gwait(j, buf, sem):
                pltpu.make_async_copy(h.at[src_v.at[j]], buf, sem).wait()

            def _scat(j, buf):
                pltpu.sync_copy(buf, acc_sh.at[ldst_v.at[j]], add=True)

            _gather(0, rows_a, sem_a)

            def pair_body(p, _):
                j0 = p * 2
                _gwait(j0, rows_a, sem_a)
                _gather(j0 + 1, rows_b, sem_b)
                _scat(j0, rows_a)
                _gwait(j0 + 1, rows_b, sem_b)
                _gather(j0 + 2, rows_a, sem_a)
                _scat(j0 + 1, rows_b)
                return _
            lax.fori_loop(0, (_NCHUNK - 1) // 2, pair_body, None)
            _gwait(_NCHUNK - 1, rows_a, sem_a)
            _scat(_NCHUNK - 1, rows_a)
            plsc.subcore_barrier()

            pltpu.sync_copy(acc_sh.at[pl.ds(s * wout, wout)],
                            out_refs[f].at[pl.ds(c * _HALF + s * wout, wout)])

            @pl.when(s == _NS - 1)
            def _(f=f):
                rem = _HALF - _NS * wout
                pltpu.sync_copy(acc_sh.at[pl.ds(_NS * wout, rem)],
                                out_refs[f].at[pl.ds(c * _HALF + _NS * wout, rem)])

    return smooth


_deg_kernel = _make_deg()
_FB = 128
_NF1 = _D1 // _FB
_smooth_d1 = _make_smooth(_FB, _NF1)
_smooth_d2 = _make_smooth(_D2, 1)


def _tc1(X, W1, b1, dega, degb):
    def body(x_ref, w_ref, b_ref, da_ref, db_ref,
             h_ref, *out_refs):
        hp_refs = out_refs[:_NF1]
        isq_ref, inv_ref = out_refs[_NF1], out_refs[_NF1 + 1]
        d1 = da_ref[...] + db_ref[...] + 1.0
        isq = lax.rsqrt(d1)
        h = jnp.dot(x_ref[...], w_ref[...],
                    preferred_element_type=jnp.float32) + b_ref[...]
        hp = h * isq
        h_ref[...] = h
        for k in range(_NF1):
            hp_refs[k][...] = hp[:, k * _FB:(k + 1) * _FB]
        isq_ref[...] = isq
        inv_ref[...] = 1.0 / d1

    return pl.pallas_call(
        body, grid=(_N // _BN,),
        in_specs=[
            pl.BlockSpec((_BN, _D1), lambda i: (i, 0)),
            pl.BlockSpec((_D1, _D1), lambda i: (0, 0)),
            pl.BlockSpec((1, _D1), lambda i: (0, 0)),
            pl.BlockSpec((_BN, 1), lambda i: (i, 0)),
            pl.BlockSpec((_BN, 1), lambda i: (i, 0)),
        ],
        out_specs=(
            [pl.BlockSpec((_BN, _D1), lambda i: (i, 0))]
            + [pl.BlockSpec((_BN, _FB), lambda i: (i, 0))] * _NF1
            + [pl.BlockSpec((_BN, 1), lambda i: (i, 0))] * 2
        ),
        out_shape=(
            [jax.ShapeDtypeStruct((_N, _D1), jnp.float32)]
            + [jax.ShapeDtypeStruct((_N, _FB), jnp.float32)] * _NF1
            + [jax.ShapeDtypeStruct((_N, 1), jnp.float32)] * 2
        ),
    )(X, W1, b1, dega, degb)


def _tc2(S1s, h1, isq, inv, W2p, b2p):
    def body(*refs):
        s1_refs = refs[:_NF1]
        (h1_ref, isq_ref, inv_ref, w_ref, b_ref,
         o1_ref, h2_ref, hp_ref) = refs[_NF1:]
        isq = isq_ref[...]
        s1 = jnp.concatenate([r[...] for r in s1_refs], axis=1)
        out1 = jnp.maximum(isq * s1 + inv_ref[...] * h1_ref[...], 0.0)
        h2 = jnp.dot(out1, w_ref[...],
                     preferred_element_type=jnp.float32) + b_ref[...]
        o1_ref[...] = out1
        h2_ref[...] = h2
        hp_ref[...] = h2 * isq

    return pl.pallas_call(
        body, grid=(_N // _BN,),
        in_specs=[pl.BlockSpec((_BN, _FB), lambda i: (i, 0))] * _NF1 + [
            pl.BlockSpec((_BN, _D1), lambda i: (i, 0)),
            pl.BlockSpec((_BN, 1), lambda i: (i, 0)),
            pl.BlockSpec((_BN, 1), lambda i: (i, 0)),
            pl.BlockSpec((_D1, _D2), lambda i: (0, 0)),
            pl.BlockSpec((1, _D2), lambda i: (0, 0)),
        ],
        out_specs=[
            pl.BlockSpec((_BN, _D1), lambda i: (i, 0)),
            pl.BlockSpec((_BN, _D2), lambda i: (i, 0)),
            pl.BlockSpec((_BN, _D2), lambda i: (i, 0)),
        ],
        out_shape=[
            jax.ShapeDtypeStruct((_N, _D1), jnp.float32),
            jax.ShapeDtypeStruct((_N, _D2), jnp.float32),
            jax.ShapeDtypeStruct((_N, _D2), jnp.float32),
        ],
    )(*S1s, h1, isq, inv, W2p, b2p)


def _tc3(S2, h2, isq, inv):
    def body(s2_ref, h2_ref, isq_ref, inv_ref, o2_ref):
        o2_ref[...] = isq_ref[...] * s2_ref[...] + inv_ref[...] * h2_ref[...]

    return pl.pallas_call(
        body, grid=(_N // _BN,),
        in_specs=[
            pl.BlockSpec((_BN, _D2), lambda i: (i, 0)),
            pl.BlockSpec((_BN, _D2), lambda i: (i, 0)),
            pl.BlockSpec((_BN, 1), lambda i: (i, 0)),
            pl.BlockSpec((_BN, 1), lambda i: (i, 0)),
        ],
        out_specs=pl.BlockSpec((_BN, _D2), lambda i: (i, 0)),
        out_shape=jax.ShapeDtypeStruct((_N, _D2), jnp.float32),
    )(S2, h2, isq, inv)


def kernel(X, edge_index, W1, b1, W2, b2):
    src = edge_index[0].reshape(_NS, _NCHUNK, _CHUNK)
    dst = edge_index[1].reshape(_NS, _NCHUNK, _CHUNK)
    dst_deg = edge_index[1].reshape(_NC * _NS, _DNCH, _DCH)

    degP = _deg_kernel(dst_deg)
    dega = degP[0, :_N, 0].reshape(_N, 1)
    degb = degP[1, :_N, 0].reshape(_N, 1)

    h1, *rest = _tc1(X, W1, b1.reshape(1, _D1), dega, degb)
    h1ps, isq, inv = rest[:_NF1], rest[_NF1], rest[_NF1 + 1]
    S1s = _smooth_d1(*h1ps, src, dst)

    W2p = jnp.pad(W2, ((0, 0), (0, _D2 - W2.shape[1])))
    b2p = jnp.pad(b2, (0, _D2 - b2.shape[0])).reshape(1, _D2)
    out1, h2, h2p = _tc2(S1s, h1, isq, inv, W2p, b2p)

    (S2,) = _smooth_d2(h2p, src, dst)
    out2p = _tc3(S2, h2, isq, inv)
    return (out1, out2p[:, :W2.shape[1]])
